# baseline (speedup 1.0000x reference)
import functools

import jax
import jax.numpy as jnp
import numpy as np
from jax import lax
from jax.experimental import pallas as pl
from jax.experimental.pallas import tpu as pltpu
from jax.experimental.pallas import tpu_sc as plsc

N = 10000
E = 320000
D = 128
D_EDGE = 16
D_DIST = 16
HID = 8
AVG_NUM_NEIGHBORS = 32.0

CH = 40
N_WORKERS = 32
EPW = E // N_WORKERS
CPW = EPW // CH
ROWS_PER_TILE = 624
ROWS_TAIL = N - 16 * ROWS_PER_TILE

_A_REP = np.kron(np.eye(HID, dtype=np.float32), np.ones((1, D_EDGE), np.float32))
_B_TILE = np.tile(np.eye(D_EDGE, dtype=np.float32), (1, HID))

_W_IDX = np.arange(D // 2)
_COL_LO = 32 * (_W_IDX // 16) + (_W_IDX % 16)
_COL_HI = _COL_LO + 16


def _pack_bf16_pair(lo, hi):
    li = jax.lax.bitcast_convert_type(lo, jnp.int32)
    hi_i = jax.lax.bitcast_convert_type(hi, jnp.int32)
    li = li + jnp.int32(0x7FFF) + ((li >> 16) & 1)
    hi_i = hi_i + jnp.int32(0x7FFF) + ((hi_i >> 16) & 1)
    return jax.lax.shift_right_logical(li, 16) | (hi_i & jnp.int32(-65536))


def _ssp(x):
    return jnp.maximum(x, 0.0) + jnp.log(1.0 + jnp.exp(-jnp.abs(x))) - np.float32(np.log(2.0))


def _x_body(nf_ref, w1_ref, x_ref):
    x_ref[...] = jnp.dot(nf_ref[...], w1_ref[...], preferred_element_type=jnp.float32)


def _g_body(edist_ref, ediff_ref, wfc1t_ref, at_ref, bt_ref,
            wblo_ref, wbhi_ref, g_ref):
    ht = _ssp(jnp.dot(wfc1t_ref[...], edist_ref[...], preferred_element_type=jnp.float32))
    pt = (jnp.dot(at_ref[...], ht, preferred_element_type=jnp.float32) * jnp.dot(
        bt_ref[...], ediff_ref[...], preferred_element_type=jnp.float32)
    ).astype(jnp.bfloat16)
    lo = jax.lax.dot_general(pt, wblo_ref[...], (((0,), (0,)), ((), ())),
                             preferred_element_type=jnp.float32)
    hi = jax.lax.dot_general(pt, wbhi_ref[...], (((0,), (0,)), ((), ())),
                             preferred_element_type=jnp.float32)
    g_ref[...] = _pack_bf16_pair(lo, hi)


def _sc_body(x_hbm, g_hbm, idx_hbm, zeros_hbm, out_hbm,
             src_ring, dst_ring, xs2, gv2, ob2, acc_sh,
             g0, g1, l0, l1, s0, s1,
             is0, is1, is2, is3, id0, id1, id2, id3):
    gsem = [g0, g1]
    lsem = [l0, l1]
    ssem = [s0, s1]
    isem_s = [is0, is1, is2, is3]
    isem_d = [id0, id1, id2, id3]
    cid = lax.axis_index("c")
    sid = lax.axis_index("s")
    wid = sid * 2 + cid

    r0 = sid * ROWS_PER_TILE
    pltpu.sync_copy(zeros_hbm.at[pl.ds(r0, ROWS_PER_TILE), :],
                    acc_sh.at[pl.ds(r0, ROWS_PER_TILE), :])

    @pl.when(sid == 15)
    def _init_tail():
        pltpu.sync_copy(zeros_hbm.at[pl.ds(N - ROWS_TAIL, ROWS_TAIL), :],
                        acc_sh.at[pl.ds(N - ROWS_TAIL, ROWS_TAIL), :])

    e0 = wid * EPW

    def issue_idx(cc, s4):
        pltpu.async_copy(idx_hbm.at[pl.ds(E + e0 + cc * CH, CH)],
                         src_ring.at[s4], isem_s[s4])
        pltpu.async_copy(idx_hbm.at[pl.ds(e0 + cc * CH, CH)],
                         dst_ring.at[s4], isem_d[s4])

    def wait_idx_s(cc, s4):
        pltpu.make_async_copy(idx_hbm.at[pl.ds(E + e0 + cc * CH, CH)],
                              src_ring.at[s4], isem_s[s4]).wait()

    def issue_data(cc, s4, pb):
        pltpu.async_copy(x_hbm.at[src_ring.at[s4]], xs2.at[pb], gsem[pb])
        pltpu.async_copy(g_hbm.at[pl.ds(e0 + cc * CH, CH), :], gv2.at[pb],
                         lsem[pb])

    for b in range(2):
        issue_idx(b, b)
    for b in range(2):
        wait_idx_s(b, b)
        issue_data(b, b, b)

    plsc.subcore_barrier()

    def quad_body(it, carry):
        for b in range(4):
            cc = it * 4 + b
            pb = b % 2
            nx = (b + 2) % 4

            @pl.when(cc < CPW)
            def _process():
                pltpu.make_async_copy(x_hbm.at[src_ring.at[b]], xs2.at[pb],
                                      gsem[pb]).wait()
                pltpu.make_async_copy(g_hbm.at[pl.ds(e0 + cc * CH, CH), :],
                                      gv2.at[pb], lsem[pb]).wait()

                @pl.when(cc >= 2)
                def _drain_prev():
                    pltpu.make_async_copy(ob2.at[pb],
                                          acc_sh.at[dst_ring.at[nx]],
                                          ssem[pb]).wait()

                @pl.when(cc + 2 < CPW)
                def _pref_idx():
                    issue_idx(cc + 2, nx)

                hi_mask = jnp.int32(-65536)

                def mul_row(i, c2):
                    for j in range(D // 32):
                        gi = gv2[pb, i, pl.ds(j * 16, 16)]
                        bc = lambda v: jax.lax.bitcast_convert_type(v, jnp.float32)
                        lo = xs2[pb, i, pl.ds(j * 32, 16)] * bc(gi << 16)
                        hi = xs2[pb, i, pl.ds(j * 32 + 16, 16)] * bc(gi & hi_mask)
                        ob2[pb, i, pl.ds(j * 32, 16)] = lo
                        ob2[pb, i, pl.ds(j * 32 + 16, 16)] = hi
                    return c2

                lax.fori_loop(0, CH, mul_row, 0)

                @pl.when(cc + 2 < CPW)
                def _pref_data():
                    wait_idx_s(cc + 2, nx)
                    issue_data(cc + 2, nx, pb)

                pltpu.make_async_copy(idx_hbm.at[pl.ds(e0 + cc * CH, CH)],
                                      dst_ring.at[b], isem_d[b]).wait()
                pltpu.async_copy(ob2.at[pb], acc_sh.at[dst_ring.at[b]],
                                 ssem[pb], add=True)
        return carry

    lax.fori_loop(0, (CPW + 3) // 4, quad_body, 0)

    for b in range(2):
        s4 = (CPW - 2 + b) % 4
        pltpu.make_async_copy(ob2.at[b], acc_sh.at[dst_ring.at[s4]],
                              ssem[b]).wait()
    plsc.subcore_barrier()

    pltpu.sync_copy(acc_sh.at[pl.ds(r0, ROWS_PER_TILE), :],
                    out_hbm.at[cid, pl.ds(r0, ROWS_PER_TILE), :])

    @pl.when(sid == 15)
    def _drain_tail():
        pltpu.sync_copy(acc_sh.at[pl.ds(N - ROWS_TAIL, ROWS_TAIL), :],
                        out_hbm.at[cid, pl.ds(N - ROWS_TAIL, ROWS_TAIL), :])


def _out_body(agg_ref, nf_ref, attr_ref, w2_ref, wsc_ref, out_ref):
    agg = agg_ref[0, :, :] + agg_ref[1, :, :]
    sc = jnp.dot(nf_ref[...] * attr_ref[...], wsc_ref[...],
                 preferred_element_type=jnp.float32)
    out_ref[...] = jnp.dot(agg, w2_ref[...], preferred_element_type=jnp.float32) + sc


def kernel(node_feat, node_attr, edge_diff_embedding, edge_dist_embedding, edge_idx,
           W1, Wfc1, Wfc2, W2, Wsc):
    f32 = jnp.float32
    inv_sqrt_d = np.float32(1.0 / np.sqrt(D))

    clo = jnp.asarray(_COL_LO)
    chi = jnp.asarray(_COL_HI)
    w1s = (W1 * inv_sqrt_d).astype(f32)
    wfc1s = (Wfc1 / np.float32(np.sqrt(D_DIST))).astype(f32)
    wbig = (Wfc2.reshape(HID, D, D_EDGE).transpose(0, 2, 1).reshape(HID * D_EDGE, D)
            / np.float32(np.sqrt(HID * D_EDGE))).astype(f32)
    w2s = (W2 * inv_sqrt_d / np.float32(np.sqrt(AVG_NUM_NEIGHBORS))).astype(f32)
    wscs = (Wsc * inv_sqrt_d).astype(f32)

    x = pl.pallas_call(
        _x_body,
        out_shape=jax.ShapeDtypeStruct((N, D), f32),
    )(node_feat, w1s)

    BE = 3200
    g = pl.pallas_call(
        _g_body,
        grid=(E // BE,),
        in_specs=[
            pl.BlockSpec((D_DIST, BE), lambda i: (0, i)),
            pl.BlockSpec((D_EDGE, BE), lambda i: (0, i)),
            pl.BlockSpec((HID, D_DIST), lambda i: (0, 0)),
            pl.BlockSpec((HID * D_EDGE, HID), lambda i: (0, 0)),
            pl.BlockSpec((HID * D_EDGE, D_EDGE), lambda i: (0, 0)),
            pl.BlockSpec((HID * D_EDGE, D // 2), lambda i: (0, 0)),
            pl.BlockSpec((HID * D_EDGE, D // 2), lambda i: (0, 0)),
        ],
        out_specs=pl.BlockSpec((BE, D // 2), lambda i: (i, 0)),
        out_shape=jax.ShapeDtypeStruct((E, D // 2), jnp.int32),
    )(edge_dist_embedding.T, edge_diff_embedding.T, wfc1s.T,
      jnp.asarray(_A_REP.T), jnp.asarray(_B_TILE.T),
      wbig[:, clo].astype(jnp.bfloat16), wbig[:, chi].astype(jnp.bfloat16))

    idxflat = jnp.reshape(edge_idx.astype(jnp.int32).T, (2 * E,))
    zeros = jnp.zeros((N, D), f32)

    sc_call = functools.partial(
        pl.kernel,
        mesh=plsc.VectorSubcoreMesh(core_axis_name="c", subcore_axis_name="s"),
        out_type=jax.ShapeDtypeStruct((2, N, D), f32),
        scratch_types=[
            pltpu.VMEM((4, CH), jnp.int32),
            pltpu.VMEM((4, CH), jnp.int32),
            pltpu.VMEM((2, CH, D), f32),
            pltpu.VMEM((2, CH, D // 2), jnp.int32),
            pltpu.VMEM((2, CH, D), f32),
            pltpu.VMEM_SHARED((N, D), f32),
        ] + [pltpu.SemaphoreType.DMA] * 14,
    )(_sc_body)
    agg2 = sc_call(x, g, idxflat, zeros)

    out = pl.pallas_call(
        _out_body,
        out_shape=jax.ShapeDtypeStruct((N, D), f32),
    )(agg2, node_feat, node_attr, w2s, wscs)
    return out


# speedup vs baseline: 1.1179x; 1.1179x over previous
import functools

import jax
import jax.numpy as jnp
import numpy as np
from jax import lax
from jax.experimental import pallas as pl
from jax.experimental.pallas import tpu as pltpu
from jax.experimental.pallas import tpu_sc as plsc

N = 10000
E = 320000
D = 128
D_EDGE = 16
D_DIST = 16
HID = 8
AVG_NUM_NEIGHBORS = 32.0

CH = 40
N_WORKERS = 32
EPW = E // N_WORKERS
CPW = EPW // CH
ROWS_PER_TILE = 624
ROWS_TAIL = N - 16 * ROWS_PER_TILE

_A_REP = np.kron(np.eye(HID, dtype=np.float32), np.ones((1, D_EDGE), np.float32))
_B_TILE = np.tile(np.eye(D_EDGE, dtype=np.float32), (1, HID))

N_HALF = 2
E_H = E // N_HALF
EPW_H = EPW // N_HALF
CPW_H = EPW_H // CH


def _ssp(x):
    return jnp.maximum(x, 0.0) + jnp.log(1.0 + jnp.exp(-jnp.abs(x))) - np.float32(np.log(2.0))


def _x_body(nf_ref, w1_ref, x_ref):
    x_ref[...] = jnp.dot(nf_ref[...], w1_ref[...], preferred_element_type=jnp.float32)


def _g_body(edist_ref, ediff_ref, wfc1t_ref, at_ref, bt_ref, wbig_ref, g_ref):
    ht = _ssp(jnp.dot(wfc1t_ref[...], edist_ref[...], preferred_element_type=jnp.float32))
    pt = jnp.dot(at_ref[...], ht, preferred_element_type=jnp.float32) * jnp.dot(
        bt_ref[...], ediff_ref[...], preferred_element_type=jnp.float32)
    g_ref[...] = jax.lax.dot_general(
        pt, wbig_ref[...], (((0,), (0,)), ((), ())),
        preferred_element_type=jnp.float32)


def _make_sc_body(half):
    ebase = half * E_H

    def _sc_body(x_hbm, g_hbm, idx_hbm, prev_hbm, out_hbm,
                 src_ring, dst_ring, xs2, gv2, ob2, acc_sh,
                 g0, g1, l0, l1, s0, s1,
                 is0, is1, is2, is3, id0, id1, id2, id3):
        gsem = [g0, g1]
        lsem = [l0, l1]
        ssem = [s0, s1]
        isem_s = [is0, is1, is2, is3]
        isem_d = [id0, id1, id2, id3]
        cid = lax.axis_index("c")
        sid = lax.axis_index("s")
        wid = sid * 2 + cid

        r0 = sid * ROWS_PER_TILE
        pltpu.sync_copy(prev_hbm.at[cid, pl.ds(r0, ROWS_PER_TILE), :],
                        acc_sh.at[pl.ds(r0, ROWS_PER_TILE), :])

        @pl.when(sid == 15)
        def _init_tail():
            pltpu.sync_copy(prev_hbm.at[cid, pl.ds(N - ROWS_TAIL, ROWS_TAIL), :],
                            acc_sh.at[pl.ds(N - ROWS_TAIL, ROWS_TAIL), :])

        e0 = ebase + wid * EPW_H
        gl0 = wid * EPW_H

        def issue_idx(cc, s4):
            pltpu.async_copy(idx_hbm.at[pl.ds(E + e0 + cc * CH, CH)],
                             src_ring.at[s4], isem_s[s4])
            pltpu.async_copy(idx_hbm.at[pl.ds(e0 + cc * CH, CH)],
                             dst_ring.at[s4], isem_d[s4])

        def wait_idx_s(cc, s4):
            pltpu.make_async_copy(idx_hbm.at[pl.ds(E + e0 + cc * CH, CH)],
                                  src_ring.at[s4], isem_s[s4]).wait()

        def issue_data(cc, s4, pb):
            pltpu.async_copy(x_hbm.at[src_ring.at[s4]], xs2.at[pb], gsem[pb])
            pltpu.async_copy(g_hbm.at[pl.ds(gl0 + cc * CH, CH), :], gv2.at[pb],
                             lsem[pb])

        for b in range(2):
            issue_idx(b, b)
        for b in range(2):
            wait_idx_s(b, b)
            issue_data(b, b, b)

        plsc.subcore_barrier()

        def quad_body(it, carry):
            for b in range(4):
                cc = it * 4 + b
                pb = b % 2
                nx = (b + 2) % 4

                @pl.when(cc < CPW_H)
                def _process():
                    pltpu.make_async_copy(x_hbm.at[src_ring.at[b]], xs2.at[pb],
                                          gsem[pb]).wait()
                    pltpu.make_async_copy(g_hbm.at[pl.ds(gl0 + cc * CH, CH), :],
                                          gv2.at[pb], lsem[pb]).wait()

                    @pl.when(cc >= 2)
                    def _drain_prev():
                        pltpu.make_async_copy(ob2.at[pb],
                                              acc_sh.at[dst_ring.at[nx]],
                                              ssem[pb]).wait()

                    @pl.when(cc + 2 < CPW_H)
                    def _pref_idx():
                        issue_idx(cc + 2, nx)

                    def mul_row(i, c2):
                        for j in range(D // 16):
                            sl = pl.ds(j * 16, 16)
                            ob2[pb, i, sl] = xs2[pb, i, sl] * gv2[pb, i, sl]
                        return c2

                    lax.fori_loop(0, CH, mul_row, 0)

                    @pl.when(cc + 2 < CPW_H)
                    def _pref_data():
                        wait_idx_s(cc + 2, nx)
                        issue_data(cc + 2, nx, pb)

                    pltpu.make_async_copy(idx_hbm.at[pl.ds(e0 + cc * CH, CH)],
                                          dst_ring.at[b], isem_d[b]).wait()
                    pltpu.async_copy(ob2.at[pb], acc_sh.at[dst_ring.at[b]],
                                     ssem[pb], add=True)
            return carry

        lax.fori_loop(0, (CPW_H + 3) // 4, quad_body, 0)

        for b in range(2):
            s4 = (CPW_H - 2 + b) % 4
            pltpu.make_async_copy(ob2.at[b], acc_sh.at[dst_ring.at[s4]],
                                  ssem[b]).wait()
        plsc.subcore_barrier()

        pltpu.sync_copy(acc_sh.at[pl.ds(r0, ROWS_PER_TILE), :],
                        out_hbm.at[cid, pl.ds(r0, ROWS_PER_TILE), :])

        @pl.when(sid == 15)
        def _drain_tail():
            pltpu.sync_copy(acc_sh.at[pl.ds(N - ROWS_TAIL, ROWS_TAIL), :],
                            out_hbm.at[cid, pl.ds(N - ROWS_TAIL, ROWS_TAIL), :])

    return _sc_body


def _out_body(agg_ref, nf_ref, attr_ref, w2_ref, wsc_ref, out_ref):
    agg = agg_ref[0, :, :] + agg_ref[1, :, :]
    sc = jnp.dot(nf_ref[...] * attr_ref[...], wsc_ref[...],
                 preferred_element_type=jnp.float32)
    out_ref[...] = jnp.dot(agg, w2_ref[...], preferred_element_type=jnp.float32) + sc


def kernel(node_feat, node_attr, edge_diff_embedding, edge_dist_embedding, edge_idx,
           W1, Wfc1, Wfc2, W2, Wsc):
    f32 = jnp.float32
    inv_sqrt_d = np.float32(1.0 / np.sqrt(D))

    w1s = (W1 * inv_sqrt_d).astype(f32)
    wfc1s = (Wfc1 / np.float32(np.sqrt(D_DIST))).astype(f32)
    wbig = (Wfc2.reshape(HID, D, D_EDGE).transpose(0, 2, 1).reshape(HID * D_EDGE, D)
            / np.float32(np.sqrt(HID * D_EDGE))).astype(f32)
    w2s = (W2 * inv_sqrt_d / np.float32(np.sqrt(AVG_NUM_NEIGHBORS))).astype(f32)
    wscs = (Wsc * inv_sqrt_d).astype(f32)

    x = pl.pallas_call(
        _x_body,
        out_shape=jax.ShapeDtypeStruct((N, D), f32),
    )(node_feat, w1s)

    BE = 3200
    BLK_H = E_H // BE

    def _g_half(half):
        return pl.pallas_call(
            _g_body,
            grid=(BLK_H,),
            in_specs=[
                pl.BlockSpec((D_DIST, BE), lambda i: (0, half * BLK_H + i)),
                pl.BlockSpec((D_EDGE, BE), lambda i: (0, half * BLK_H + i)),
                pl.BlockSpec((HID, D_DIST), lambda i: (0, 0)),
                pl.BlockSpec((HID * D_EDGE, HID), lambda i: (0, 0)),
                pl.BlockSpec((HID * D_EDGE, D_EDGE), lambda i: (0, 0)),
                pl.BlockSpec((HID * D_EDGE, D), lambda i: (0, 0)),
            ],
            out_specs=pl.BlockSpec((BE, D), lambda i: (i, 0)),
            out_shape=jax.ShapeDtypeStruct((E_H, D), f32),
        )(edge_dist_embedding.T, edge_diff_embedding.T, wfc1s.T,
          jnp.asarray(_A_REP.T), jnp.asarray(_B_TILE.T), wbig)

    g_a = _g_half(0)
    g_b = _g_half(1)

    idxflat = jnp.reshape(edge_idx.astype(jnp.int32).T, (2 * E,))
    zeros = jnp.zeros((2, N, D), f32)

    def _sc_half(half):
        return functools.partial(
            pl.kernel,
            mesh=plsc.VectorSubcoreMesh(core_axis_name="c", subcore_axis_name="s"),
            out_type=jax.ShapeDtypeStruct((2, N, D), f32),
            scratch_types=[
                pltpu.VMEM((4, CH), jnp.int32),
                pltpu.VMEM((4, CH), jnp.int32),
                pltpu.VMEM((2, CH, D), f32),
                pltpu.VMEM((2, CH, D), f32),
                pltpu.VMEM((2, CH, D), f32),
                pltpu.VMEM_SHARED((N, D), f32),
            ] + [pltpu.SemaphoreType.DMA] * 14,
        )(_make_sc_body(half))

    agg_a = _sc_half(0)(x, g_a, idxflat, zeros)
    agg2 = _sc_half(1)(x, g_b, idxflat, agg_a)

    out = pl.pallas_call(
        _out_body,
        out_shape=jax.ShapeDtypeStruct((N, D), f32),
    )(agg2, node_feat, node_attr, w2s, wscs)
    return out


# speedup vs baseline: 1.1445x; 1.0238x over previous
import functools

import jax
import jax.numpy as jnp
import numpy as np
from jax import lax
from jax.experimental import pallas as pl
from jax.experimental.pallas import tpu as pltpu
from jax.experimental.pallas import tpu_sc as plsc

N = 10000
E = 320000
D = 128
D_EDGE = 16
D_DIST = 16
HID = 8
AVG_NUM_NEIGHBORS = 32.0

CH = 40
N_WORKERS = 32
EPW = E // N_WORKERS
CPW = EPW // CH
ROWS_PER_TILE = 624
ROWS_TAIL = N - 16 * ROWS_PER_TILE

_A_REP = np.kron(np.eye(HID, dtype=np.float32), np.ones((1, D_EDGE), np.float32))
_B_TILE = np.tile(np.eye(D_EDGE, dtype=np.float32), (1, HID))

EPW_A = 2400
EPW_B = EPW - EPW_A
E_A = N_WORKERS * EPW_A
E_B = E - E_A
CPW_A = EPW_A // CH
CPW_B = EPW_B // CH


def _ssp(x):
    return jnp.maximum(x, 0.0) + jnp.log(1.0 + jnp.exp(-jnp.abs(x))) - np.float32(np.log(2.0))


def _x_body(nf_ref, w1_ref, x_ref):
    x_ref[...] = jnp.dot(nf_ref[...], w1_ref[...], preferred_element_type=jnp.float32)


def _g_body(edist_ref, ediff_ref, wfc1t_ref, at_ref, bt_ref, wbig_ref, g_ref):
    ht = _ssp(jnp.dot(wfc1t_ref[...], edist_ref[...], preferred_element_type=jnp.float32))
    pt = jnp.dot(at_ref[...], ht, preferred_element_type=jnp.float32) * jnp.dot(
        bt_ref[...], ediff_ref[...], preferred_element_type=jnp.float32)
    g_ref[...] = jax.lax.dot_general(
        pt, wbig_ref[...], (((0,), (0,)), ((), ())),
        preferred_element_type=jnp.float32)


def _make_sc_body(ebase, epw, cpw):
    def _sc_body(x_hbm, g_hbm, idx_hbm, prev_hbm, out_hbm,
                 src_ring, dst_ring, xs2, gv2, ob2, acc_sh,
                 g0, g1, l0, l1, s0, s1,
                 is0, is1, is2, is3, id0, id1, id2, id3):
        gsem = [g0, g1]
        lsem = [l0, l1]
        ssem = [s0, s1]
        isem_s = [is0, is1, is2, is3]
        isem_d = [id0, id1, id2, id3]
        cid = lax.axis_index("c")
        sid = lax.axis_index("s")
        wid = sid * 2 + cid

        r0 = sid * ROWS_PER_TILE
        pltpu.sync_copy(prev_hbm.at[cid, pl.ds(r0, ROWS_PER_TILE), :],
                        acc_sh.at[pl.ds(r0, ROWS_PER_TILE), :])

        @pl.when(sid == 15)
        def _init_tail():
            pltpu.sync_copy(prev_hbm.at[cid, pl.ds(N - ROWS_TAIL, ROWS_TAIL), :],
                            acc_sh.at[pl.ds(N - ROWS_TAIL, ROWS_TAIL), :])

        e0 = ebase + wid * epw
        gl0 = wid * epw

        def issue_idx(cc, s4):
            pltpu.async_copy(idx_hbm.at[pl.ds(E + e0 + cc * CH, CH)],
                             src_ring.at[s4], isem_s[s4])
            pltpu.async_copy(idx_hbm.at[pl.ds(e0 + cc * CH, CH)],
                             dst_ring.at[s4], isem_d[s4])

        def wait_idx_s(cc, s4):
            pltpu.make_async_copy(idx_hbm.at[pl.ds(E + e0 + cc * CH, CH)],
                                  src_ring.at[s4], isem_s[s4]).wait()

        def issue_data(cc, s4, pb):
            pltpu.async_copy(x_hbm.at[src_ring.at[s4]], xs2.at[pb], gsem[pb])
            pltpu.async_copy(g_hbm.at[pl.ds(gl0 + cc * CH, CH), :], gv2.at[pb],
                             lsem[pb])

        for b in range(2):
            issue_idx(b, b)
        for b in range(2):
            wait_idx_s(b, b)
            issue_data(b, b, b)

        plsc.subcore_barrier()

        def quad_body(it, carry):
            for b in range(4):
                cc = it * 4 + b
                pb = b % 2
                nx = (b + 2) % 4

                @pl.when(cc < cpw)
                def _process():
                    pltpu.make_async_copy(x_hbm.at[src_ring.at[b]], xs2.at[pb],
                                          gsem[pb]).wait()
                    pltpu.make_async_copy(g_hbm.at[pl.ds(gl0 + cc * CH, CH), :],
                                          gv2.at[pb], lsem[pb]).wait()

                    @pl.when(cc >= 2)
                    def _drain_prev():
                        pltpu.make_async_copy(ob2.at[pb],
                                              acc_sh.at[dst_ring.at[nx]],
                                              ssem[pb]).wait()

                    @pl.when(cc + 2 < cpw)
                    def _pref_idx():
                        issue_idx(cc + 2, nx)

                    def mul_row(i, c2):
                        for j in range(D // 16):
                            sl = pl.ds(j * 16, 16)
                            ob2[pb, i, sl] = xs2[pb, i, sl] * gv2[pb, i, sl]
                        return c2

                    lax.fori_loop(0, CH, mul_row, 0)

                    @pl.when(cc + 2 < cpw)
                    def _pref_data():
                        wait_idx_s(cc + 2, nx)
                        issue_data(cc + 2, nx, pb)

                    pltpu.make_async_copy(idx_hbm.at[pl.ds(e0 + cc * CH, CH)],
                                          dst_ring.at[b], isem_d[b]).wait()
                    pltpu.async_copy(ob2.at[pb], acc_sh.at[dst_ring.at[b]],
                                     ssem[pb], add=True)
            return carry

        lax.fori_loop(0, (cpw + 3) // 4, quad_body, 0)

        for b in range(2):
            s4 = (cpw - 2 + b) % 4
            pltpu.make_async_copy(ob2.at[b], acc_sh.at[dst_ring.at[s4]],
                                  ssem[b]).wait()
        plsc.subcore_barrier()

        pltpu.sync_copy(acc_sh.at[pl.ds(r0, ROWS_PER_TILE), :],
                        out_hbm.at[cid, pl.ds(r0, ROWS_PER_TILE), :])

        @pl.when(sid == 15)
        def _drain_tail():
            pltpu.sync_copy(acc_sh.at[pl.ds(N - ROWS_TAIL, ROWS_TAIL), :],
                            out_hbm.at[cid, pl.ds(N - ROWS_TAIL, ROWS_TAIL), :])

    return _sc_body


def _out_body(agg_ref, nf_ref, attr_ref, w2_ref, wsc_ref, out_ref):
    agg = agg_ref[0, :, :] + agg_ref[1, :, :]
    sc = jnp.dot(nf_ref[...] * attr_ref[...], wsc_ref[...],
                 preferred_element_type=jnp.float32)
    out_ref[...] = jnp.dot(agg, w2_ref[...], preferred_element_type=jnp.float32) + sc


def kernel(node_feat, node_attr, edge_diff_embedding, edge_dist_embedding, edge_idx,
           W1, Wfc1, Wfc2, W2, Wsc):
    f32 = jnp.float32
    inv_sqrt_d = np.float32(1.0 / np.sqrt(D))

    w1s = (W1 * inv_sqrt_d).astype(f32)
    wfc1s = (Wfc1 / np.float32(np.sqrt(D_DIST))).astype(f32)
    wbig = (Wfc2.reshape(HID, D, D_EDGE).transpose(0, 2, 1).reshape(HID * D_EDGE, D)
            / np.float32(np.sqrt(HID * D_EDGE))).astype(f32)
    w2s = (W2 * inv_sqrt_d / np.float32(np.sqrt(AVG_NUM_NEIGHBORS))).astype(f32)
    wscs = (Wsc * inv_sqrt_d).astype(f32)

    x = pl.pallas_call(
        _x_body,
        out_shape=jax.ShapeDtypeStruct((N, D), f32),
    )(node_feat, w1s)

    BE = 3200

    def _g_part(blk0, ne):
        return pl.pallas_call(
            _g_body,
            grid=(ne // BE,),
            in_specs=[
                pl.BlockSpec((D_DIST, BE), lambda i: (0, blk0 + i)),
                pl.BlockSpec((D_EDGE, BE), lambda i: (0, blk0 + i)),
                pl.BlockSpec((HID, D_DIST), lambda i: (0, 0)),
                pl.BlockSpec((HID * D_EDGE, HID), lambda i: (0, 0)),
                pl.BlockSpec((HID * D_EDGE, D_EDGE), lambda i: (0, 0)),
                pl.BlockSpec((HID * D_EDGE, D), lambda i: (0, 0)),
            ],
            out_specs=pl.BlockSpec((BE, D), lambda i: (i, 0)),
            out_shape=jax.ShapeDtypeStruct((ne, D), f32),
        )(edge_dist_embedding.T, edge_diff_embedding.T, wfc1s.T,
          jnp.asarray(_A_REP.T), jnp.asarray(_B_TILE.T), wbig)

    g_a = _g_part(0, E_A)
    g_b = _g_part(E_A // BE, E_B)

    idxflat = jnp.reshape(edge_idx.astype(jnp.int32).T, (2 * E,))
    zeros = jnp.zeros((2, N, D), f32)

    def _sc_part(ebase, epw, cpw):
        return functools.partial(
            pl.kernel,
            mesh=plsc.VectorSubcoreMesh(core_axis_name="c", subcore_axis_name="s"),
            out_type=jax.ShapeDtypeStruct((2, N, D), f32),
            scratch_types=[
                pltpu.VMEM((4, CH), jnp.int32),
                pltpu.VMEM((4, CH), jnp.int32),
                pltpu.VMEM((2, CH, D), f32),
                pltpu.VMEM((2, CH, D), f32),
                pltpu.VMEM((2, CH, D), f32),
                pltpu.VMEM_SHARED((N, D), f32),
            ] + [pltpu.SemaphoreType.DMA] * 14,
        )(_make_sc_body(ebase, epw, cpw))

    agg_a = _sc_part(0, EPW_A, CPW_A)(x, g_a, idxflat, zeros)
    agg2 = _sc_part(E_A, EPW_B, CPW_B)(x, g_b, idxflat, agg_a)

    out = pl.pallas_call(
        _out_body,
        out_shape=jax.ShapeDtypeStruct((N, D), f32),
    )(agg2, node_feat, node_attr, w2s, wscs)
    return out


# speedup vs baseline: 1.1898x; 1.0396x over previous
import functools

import jax
import jax.numpy as jnp
import numpy as np
from jax import lax
from jax.experimental import pallas as pl
from jax.experimental.pallas import tpu as pltpu
from jax.experimental.pallas import tpu_sc as plsc

N = 10000
E = 320000
D = 128
D_EDGE = 16
D_DIST = 16
HID = 8
AVG_NUM_NEIGHBORS = 32.0

CH = 40
N_WORKERS = 32
EPW = E // N_WORKERS
CPW = EPW // CH
ROWS_PER_TILE = 624
ROWS_TAIL = N - 16 * ROWS_PER_TILE

_A_REP = np.kron(np.eye(HID, dtype=np.float32), np.ones((1, D_EDGE), np.float32))
_B_TILE = np.tile(np.eye(D_EDGE, dtype=np.float32), (1, HID))

EPW_A = 3400
EPW_B = EPW - EPW_A
E_A = N_WORKERS * EPW_A
E_B = E - E_A
CPW_A = EPW_A // CH
CPW_B = EPW_B // CH


def _ssp(x):
    return jnp.maximum(x, 0.0) + jnp.log(1.0 + jnp.exp(-jnp.abs(x))) - np.float32(np.log(2.0))


def _x_body(nf_ref, w1_ref, x_ref):
    x_ref[...] = jnp.dot(nf_ref[...], w1_ref[...], preferred_element_type=jnp.float32)


def _g_body(edist_ref, ediff_ref, wfc1t_ref, at_ref, bt_ref, wbig_ref, g_ref):
    ht = _ssp(jnp.dot(wfc1t_ref[...], edist_ref[...], preferred_element_type=jnp.float32))
    pt = jnp.dot(at_ref[...], ht, preferred_element_type=jnp.float32) * jnp.dot(
        bt_ref[...], ediff_ref[...], preferred_element_type=jnp.float32)
    g_ref[...] = jax.lax.dot_general(
        pt, wbig_ref[...], (((0,), (0,)), ((), ())),
        preferred_element_type=jnp.float32)


def _make_sc_body(ebase, epw, cpw):
    def _sc_body(x_hbm, g_hbm, idx_hbm, prev_hbm, out_hbm,
                 src_ring, dst_ring, xs2, gv2, ob2, acc_sh,
                 g0, g1, l0, l1, s0, s1,
                 is0, is1, is2, is3, id0, id1, id2, id3):
        gsem = [g0, g1]
        lsem = [l0, l1]
        ssem = [s0, s1]
        isem_s = [is0, is1, is2, is3]
        isem_d = [id0, id1, id2, id3]
        cid = lax.axis_index("c")
        sid = lax.axis_index("s")
        wid = sid * 2 + cid

        r0 = sid * ROWS_PER_TILE
        pltpu.sync_copy(prev_hbm.at[cid, pl.ds(r0, ROWS_PER_TILE), :],
                        acc_sh.at[pl.ds(r0, ROWS_PER_TILE), :])

        @pl.when(sid == 15)
        def _init_tail():
            pltpu.sync_copy(prev_hbm.at[cid, pl.ds(N - ROWS_TAIL, ROWS_TAIL), :],
                            acc_sh.at[pl.ds(N - ROWS_TAIL, ROWS_TAIL), :])

        e0 = ebase + wid * epw
        gl0 = wid * epw

        def issue_idx(cc, s4):
            pltpu.async_copy(idx_hbm.at[pl.ds(E + e0 + cc * CH, CH)],
                             src_ring.at[s4], isem_s[s4])
            pltpu.async_copy(idx_hbm.at[pl.ds(e0 + cc * CH, CH)],
                             dst_ring.at[s4], isem_d[s4])

        def wait_idx_s(cc, s4):
            pltpu.make_async_copy(idx_hbm.at[pl.ds(E + e0 + cc * CH, CH)],
                                  src_ring.at[s4], isem_s[s4]).wait()

        def issue_data(cc, s4, pb):
            pltpu.async_copy(x_hbm.at[src_ring.at[s4]], xs2.at[pb], gsem[pb])
            pltpu.async_copy(g_hbm.at[pl.ds(gl0 + cc * CH, CH), :], gv2.at[pb],
                             lsem[pb])

        for b in range(2):
            issue_idx(b, b)
        for b in range(2):
            wait_idx_s(b, b)
            issue_data(b, b, b)

        plsc.subcore_barrier()

        def quad_body(it, carry):
            for b in range(4):
                cc = it * 4 + b
                pb = b % 2
                nx = (b + 2) % 4

                @pl.when(cc < cpw)
                def _process():
                    pltpu.make_async_copy(x_hbm.at[src_ring.at[b]], xs2.at[pb],
                                          gsem[pb]).wait()
                    pltpu.make_async_copy(g_hbm.at[pl.ds(gl0 + cc * CH, CH), :],
                                          gv2.at[pb], lsem[pb]).wait()

                    @pl.when(cc >= 2)
                    def _drain_prev():
                        pltpu.make_async_copy(ob2.at[pb],
                                              acc_sh.at[dst_ring.at[nx]],
                                              ssem[pb]).wait()

                    @pl.when(cc + 2 < cpw)
                    def _pref_idx():
                        issue_idx(cc + 2, nx)

                    def mul_row(i, c2):
                        for j in range(D // 16):
                            sl = pl.ds(j * 16, 16)
                            ob2[pb, i, sl] = xs2[pb, i, sl] * gv2[pb, i, sl]
                        return c2

                    lax.fori_loop(0, CH, mul_row, 0)

                    @pl.when(cc + 2 < cpw)
                    def _pref_data():
                        wait_idx_s(cc + 2, nx)
                        issue_data(cc + 2, nx, pb)

                    pltpu.make_async_copy(idx_hbm.at[pl.ds(e0 + cc * CH, CH)],
                                          dst_ring.at[b], isem_d[b]).wait()
                    pltpu.async_copy(ob2.at[pb], acc_sh.at[dst_ring.at[b]],
                                     ssem[pb], add=True)
            return carry

        lax.fori_loop(0, (cpw + 3) // 4, quad_body, 0)

        for b in range(2):
            s4 = (cpw - 2 + b) % 4
            pltpu.make_async_copy(ob2.at[b], acc_sh.at[dst_ring.at[s4]],
                                  ssem[b]).wait()
        plsc.subcore_barrier()

        pltpu.sync_copy(acc_sh.at[pl.ds(r0, ROWS_PER_TILE), :],
                        out_hbm.at[cid, pl.ds(r0, ROWS_PER_TILE), :])

        @pl.when(sid == 15)
        def _drain_tail():
            pltpu.sync_copy(acc_sh.at[pl.ds(N - ROWS_TAIL, ROWS_TAIL), :],
                            out_hbm.at[cid, pl.ds(N - ROWS_TAIL, ROWS_TAIL), :])

    return _sc_body


def _out_body(agg_ref, nf_ref, attr_ref, w2_ref, wsc_ref, out_ref):
    agg = agg_ref[0, :, :] + agg_ref[1, :, :]
    sc = jnp.dot(nf_ref[...] * attr_ref[...], wsc_ref[...],
                 preferred_element_type=jnp.float32)
    out_ref[...] = jnp.dot(agg, w2_ref[...], preferred_element_type=jnp.float32) + sc


def kernel(node_feat, node_attr, edge_diff_embedding, edge_dist_embedding, edge_idx,
           W1, Wfc1, Wfc2, W2, Wsc):
    f32 = jnp.float32
    inv_sqrt_d = np.float32(1.0 / np.sqrt(D))

    w1s = (W1 * inv_sqrt_d).astype(f32)
    wfc1s = (Wfc1 / np.float32(np.sqrt(D_DIST))).astype(f32)
    wbig = (Wfc2.reshape(HID, D, D_EDGE).transpose(0, 2, 1).reshape(HID * D_EDGE, D)
            / np.float32(np.sqrt(HID * D_EDGE))).astype(f32)
    w2s = (W2 * inv_sqrt_d / np.float32(np.sqrt(AVG_NUM_NEIGHBORS))).astype(f32)
    wscs = (Wsc * inv_sqrt_d).astype(f32)

    x = pl.pallas_call(
        _x_body,
        out_shape=jax.ShapeDtypeStruct((N, D), f32),
    )(node_feat, w1s)

    BE = 3200

    def _g_part(blk0, ne):
        return pl.pallas_call(
            _g_body,
            grid=(ne // BE,),
            in_specs=[
                pl.BlockSpec((D_DIST, BE), lambda i: (0, blk0 + i)),
                pl.BlockSpec((D_EDGE, BE), lambda i: (0, blk0 + i)),
                pl.BlockSpec((HID, D_DIST), lambda i: (0, 0)),
                pl.BlockSpec((HID * D_EDGE, HID), lambda i: (0, 0)),
                pl.BlockSpec((HID * D_EDGE, D_EDGE), lambda i: (0, 0)),
                pl.BlockSpec((HID * D_EDGE, D), lambda i: (0, 0)),
            ],
            out_specs=pl.BlockSpec((BE, D), lambda i: (i, 0)),
            out_shape=jax.ShapeDtypeStruct((ne, D), f32),
        )(edge_dist_embedding.T, edge_diff_embedding.T, wfc1s.T,
          jnp.asarray(_A_REP.T), jnp.asarray(_B_TILE.T), wbig)

    g_a = _g_part(0, E_A)
    g_b = _g_part(E_A // BE, E_B)

    idxflat = jnp.reshape(edge_idx.astype(jnp.int32).T, (2 * E,))
    zeros = jnp.zeros((2, N, D), f32)

    def _sc_part(ebase, epw, cpw):
        return functools.partial(
            pl.kernel,
            mesh=plsc.VectorSubcoreMesh(core_axis_name="c", subcore_axis_name="s"),
            out_type=jax.ShapeDtypeStruct((2, N, D), f32),
            scratch_types=[
                pltpu.VMEM((4, CH), jnp.int32),
                pltpu.VMEM((4, CH), jnp.int32),
                pltpu.VMEM((2, CH, D), f32),
                pltpu.VMEM((2, CH, D), f32),
                pltpu.VMEM((2, CH, D), f32),
                pltpu.VMEM_SHARED((N, D), f32),
            ] + [pltpu.SemaphoreType.DMA] * 14,
        )(_make_sc_body(ebase, epw, cpw))

    agg_a = _sc_part(0, EPW_A, CPW_A)(x, g_a, idxflat, zeros)
    agg2 = _sc_part(E_A, EPW_B, CPW_B)(x, g_b, idxflat, agg_a)

    out = pl.pallas_call(
        _out_body,
        out_shape=jax.ShapeDtypeStruct((N, D), f32),
    )(agg2, node_feat, node_attr, w2s, wscs)
    return out
